# Initial kernel scaffold; baseline (speedup 1.0000x reference)
import jax
import jax.numpy as jnp
from jax.experimental import pallas as pl


def kernel(nf, edge_index, edge_type, node_type, params):
    raise NotImplementedError("write your pallas kernel here")


# speedup vs baseline: 8.7410x; 8.7410x over previous
import jax
import jax.numpy as jnp
from jax import lax
from jax.experimental import pallas as pl
from jax.experimental.pallas import tpu as pltpu
from jax.experimental.pallas import tpu_sc as plsc

N_NODES = 10000
N_EDGES = 320000
ET = 3
H = 32
NC, NS = 2, 16
NW = NC * NS
ROWS = ET * N_NODES
ROWS_PAD = 30720
RPT = ROWS_PAD // NS
ZROWS = 120
CHUNK = 80
EPW = N_EDGES // NW
NCHUNK = EPW // CHUNK
NB = 10
BN = N_NODES // NB


def _sc_edge_pass(with_count):
    def body(ps, pd, srcr, dstr, etr, *rest):
        if with_count:
            (acc_out, cnt_out, srcv, dstv, etv, idxs, idxd, av, bv, hv,
             stage, sem1, sem2, acc_s, onesv, cstage, cnt_s) = rest
        else:
            (acc_out, srcv, dstv, etv, idxs, idxd, av, bv, hv,
             stage, sem1, sem2, acc_s) = rest

        cid = lax.axis_index("c")
        sid = lax.axis_index("s")
        wid = sid * NC + cid
        tbase = sid * RPT
        zvec = jnp.zeros((16,), jnp.float32)

        def zrow(i, _):
            stage[i, pl.ds(0, 16)] = zvec
            stage[i, pl.ds(16, 16)] = zvec
            if with_count:
                cstage[i, pl.ds(0, 16)] = zvec
            return 0
        lax.fori_loop(0, ZROWS, zrow, 0)

        def zcp(j, _):
            b = tbase + j * ZROWS
            pltpu.sync_copy(stage, acc_s.at[pl.ds(b, ZROWS)])
            if with_count:
                pltpu.sync_copy(cstage, cnt_s.at[pl.ds(b, ZROWS)])
            return 0
        lax.fori_loop(0, NS, zcp, 0)

        if with_count:
            iot = lax.iota(jnp.int32, 16)
            onevec = (1 - jnp.minimum(iot, 1)).astype(jnp.float32)

            def orow(i, _):
                onesv[i, pl.ds(0, 16)] = onevec
                return 0
            lax.fori_loop(0, CHUNK, orow, 0)

        plsc.subcore_barrier()

        ebase = wid * EPW

        def chunk_body(j, _):
            off = ebase + j * CHUNK
            pltpu.sync_copy(srcr.at[pl.ds(off, CHUNK)], srcv)
            pltpu.sync_copy(dstr.at[pl.ds(off, CHUNK)], dstv)
            pltpu.sync_copy(etr.at[pl.ds(off, CHUNK)], etv)
            for i in range(CHUNK // 16):
                sl = pl.ds(i * 16, 16)
                ebias = etv[sl] * N_NODES
                idxs[sl] = ebias + srcv[sl]
                idxd[sl] = ebias + dstv[sl]
            cp1 = pltpu.async_copy(ps.at[idxs], av, sem1)
            cp2 = pltpu.async_copy(pd.at[idxd], bv, sem2)
            cp1.wait()
            cp2.wait()
            for i in range(CHUNK):
                for h0 in (0, 16):
                    sl = pl.ds(h0, 16)
                    hv[i, sl] = jnp.maximum(av[i, sl] + bv[i, sl], zvec)
            pltpu.sync_copy(hv, acc_s.at[idxd], add=True)
            if with_count:
                pltpu.sync_copy(onesv, cnt_s.at[idxd], add=True)
            return 0
        lax.fori_loop(0, NCHUNK, chunk_body, 0)

        plsc.subcore_barrier()

        def wout(j, _):
            b = tbase + j * ZROWS
            pltpu.sync_copy(acc_s.at[pl.ds(b, ZROWS)], stage)
            pltpu.sync_copy(stage, acc_out.at[pl.ds(cid * ROWS_PAD + b, ZROWS)])
            if with_count:
                pltpu.sync_copy(cnt_s.at[pl.ds(b, ZROWS)], cstage)
                pltpu.sync_copy(cstage, cnt_out.at[pl.ds(cid * ROWS_PAD + b, ZROWS)])
            return 0
        lax.fori_loop(0, NS, wout, 0)

    outs = [jax.ShapeDtypeStruct((NC * ROWS_PAD, H), jnp.float32)]
    scratch = [
        pltpu.VMEM((CHUNK,), jnp.int32),
        pltpu.VMEM((CHUNK,), jnp.int32),
        pltpu.VMEM((CHUNK,), jnp.int32),
        pltpu.VMEM((CHUNK,), jnp.int32),
        pltpu.VMEM((CHUNK,), jnp.int32),
        pltpu.VMEM((CHUNK, H), jnp.float32),
        pltpu.VMEM((CHUNK, H), jnp.float32),
        pltpu.VMEM((CHUNK, H), jnp.float32),
        pltpu.VMEM((ZROWS, H), jnp.float32),
        pltpu.SemaphoreType.DMA,
        pltpu.SemaphoreType.DMA,
        pltpu.VMEM_SHARED((ROWS_PAD, H), jnp.float32),
    ]
    if with_count:
        outs.append(jax.ShapeDtypeStruct((NC * ROWS_PAD, 16), jnp.float32))
        scratch += [
            pltpu.VMEM((CHUNK, 16), jnp.float32),
            pltpu.VMEM((ZROWS, 16), jnp.float32),
            pltpu.VMEM_SHARED((ROWS_PAD, 16), jnp.float32),
        ]

    mesh = plsc.VectorSubcoreMesh(core_axis_name="c", subcore_axis_name="s")
    return pl.kernel(
        body,
        out_type=tuple(outs) if with_count else outs[0],
        scratch_types=scratch,
        mesh=mesh,
        compiler_params=pltpu.CompilerParams(use_tc_tiling_on_sc=False),
    )


def _tc_pre(nf, ws, wd, b1):
    din = nf.shape[1]

    def body(nf_ref, ws_ref, wd_ref, b1_ref, ps_ref, pd_ref):
        x = nf_ref[...]
        for e in range(ET):
            ps_ref[e] = jnp.dot(x, ws_ref[e], preferred_element_type=jnp.float32)
            pd_ref[e] = (jnp.dot(x, wd_ref[e], preferred_element_type=jnp.float32)
                         + b1_ref[e])

    return pl.pallas_call(
        body,
        grid=(NB,),
        in_specs=[
            pl.BlockSpec((BN, din), lambda g: (g, 0)),
            pl.BlockSpec((ET, din, H), lambda g: (0, 0, 0)),
            pl.BlockSpec((ET, din, H), lambda g: (0, 0, 0)),
            pl.BlockSpec((ET, 1, H), lambda g: (0, 0, 0)),
        ],
        out_specs=[
            pl.BlockSpec((ET, BN, H), lambda g: (0, g, 0)),
            pl.BlockSpec((ET, BN, H), lambda g: (0, g, 0)),
        ],
        out_shape=[
            jax.ShapeDtypeStruct((ET, N_NODES, H), jnp.float32),
            jax.ShapeDtypeStruct((ET, N_NODES, H), jnp.float32),
        ],
    )(nf, ws, wd, b1)


def _node_update(acc_ref, cnt_ref, nt_ref, w2_ref, b2_ref,
                 w1n_ref, b1n_ref, w2n_ref, b2n_ref):
    msgs = []
    for e in range(ET):
        s = acc_ref[0, e] + acc_ref[1, e]
        c = jnp.sum(cnt_ref[0, e] + cnt_ref[1, e], axis=-1,
                    keepdims=True)
        m = s / jnp.maximum(c, 1.0)
        ind = (c > 0.0).astype(jnp.float32)
        msgs.append(jnp.dot(m, w2_ref[e], preferred_element_type=jnp.float32)
                    + b2_ref[e] * ind)
    msg = jnp.concatenate(msgs, axis=-1)
    outs = []
    for i in range(2):
        hh = jnp.maximum(
            jnp.dot(msg, w1n_ref[i], preferred_element_type=jnp.float32)
            + b1n_ref[i], 0.0)
        outs.append(jnp.dot(hh, w2n_ref[i], preferred_element_type=jnp.float32)
                    + b2n_ref[i])
    ntv = nt_ref[...]
    return jnp.where(ntv == 1, outs[1], outs[0])


def _head_specs(dout):
    return [
        pl.BlockSpec((NC, ET, BN, H), lambda g: (0, 0, g, 0)),
        pl.BlockSpec((NC, ET, BN, 16), lambda g: (0, 0, g, 0)),
        pl.BlockSpec((BN, 1), lambda g: (g, 0)),
        pl.BlockSpec((ET, H, H), lambda g: (0, 0, 0)),
        pl.BlockSpec((ET, 1, H), lambda g: (0, 0, 0)),
        pl.BlockSpec((2, ET * H, dout), lambda g: (0, 0, 0)),
        pl.BlockSpec((2, 1, dout), lambda g: (0, 0, 0)),
        pl.BlockSpec((2, dout, dout), lambda g: (0, 0, 0)),
        pl.BlockSpec((2, 1, dout), lambda g: (0, 0, 0)),
    ]


def _tc_mid(acc, cnt, nt2, w2, b2, w1n, b1n, w2n, b2n, ws1, wd1, b11):
    dout = w1n.shape[2]

    def body(acc_ref, cnt_ref, nt_ref, w2_ref, b2_ref, w1n_ref, b1n_ref,
             w2n_ref, b2n_ref, ws1_ref, wd1_ref, b11_ref, ps_ref, pd_ref):
        x = _node_update(acc_ref, cnt_ref, nt_ref, w2_ref, b2_ref,
                         w1n_ref, b1n_ref, w2n_ref, b2n_ref)
        for e in range(ET):
            ps_ref[e] = jnp.dot(x, ws1_ref[e], preferred_element_type=jnp.float32)
            pd_ref[e] = (jnp.dot(x, wd1_ref[e], preferred_element_type=jnp.float32)
                         + b11_ref[e])

    return pl.pallas_call(
        body,
        grid=(NB,),
        in_specs=_head_specs(dout) + [
            pl.BlockSpec((ET, dout, H), lambda g: (0, 0, 0)),
            pl.BlockSpec((ET, dout, H), lambda g: (0, 0, 0)),
            pl.BlockSpec((ET, 1, H), lambda g: (0, 0, 0)),
        ],
        out_specs=[
            pl.BlockSpec((ET, BN, H), lambda g: (0, g, 0)),
            pl.BlockSpec((ET, BN, H), lambda g: (0, g, 0)),
        ],
        out_shape=[
            jax.ShapeDtypeStruct((ET, N_NODES, H), jnp.float32),
            jax.ShapeDtypeStruct((ET, N_NODES, H), jnp.float32),
        ],
    )(acc, cnt, nt2, w2, b2, w1n, b1n, w2n, b2n, ws1, wd1, b11)


def _tc_post(acc, cnt, nt2, w2, b2, w1n, b1n, w2n, b2n):
    dout = w1n.shape[2]

    def body(acc_ref, cnt_ref, nt_ref, w2_ref, b2_ref, w1n_ref, b1n_ref,
             w2n_ref, b2n_ref, out_ref):
        out_ref[...] = _node_update(acc_ref, cnt_ref, nt_ref, w2_ref, b2_ref,
                                    w1n_ref, b1n_ref, w2n_ref, b2n_ref)

    return pl.pallas_call(
        body,
        grid=(NB,),
        in_specs=_head_specs(dout),
        out_specs=pl.BlockSpec((BN, dout), lambda g: (g, 0)),
        out_shape=jax.ShapeDtypeStruct((N_NODES, dout), jnp.float32),
    )(acc, cnt, nt2, w2, b2, w1n, b1n, w2n, b2n)


def kernel(nf, edge_index, edge_type, node_type, params):
    src = edge_index[0].astype(jnp.int32)
    dst = edge_index[1].astype(jnp.int32)
    et = edge_type.astype(jnp.int32)
    nt2 = node_type.astype(jnp.int32).reshape(N_NODES, 1)

    l0, l1 = params
    din0 = nf.shape[1]
    ws0 = jnp.stack([p["W1"][:din0] for p in l0["edge"]])
    wd0 = jnp.stack([p["W1"][din0:] for p in l0["edge"]])
    b10 = jnp.stack([p["b1"] for p in l0["edge"]]).reshape(ET, 1, H)
    w20 = jnp.stack([p["W2"] for p in l0["edge"]])
    b20 = jnp.stack([p["b2"] for p in l0["edge"]]).reshape(ET, 1, H)
    w1n0 = jnp.stack([p["W1"] for p in l0["node"]])
    w2n0 = jnp.stack([p["W2"] for p in l0["node"]])
    dm0 = w1n0.shape[2]
    b1n0 = jnp.stack([p["b1"] for p in l0["node"]]).reshape(2, 1, dm0)
    b2n0 = jnp.stack([p["b2"] for p in l0["node"]]).reshape(2, 1, dm0)
    din1 = dm0
    ws1 = jnp.stack([p["W1"][:din1] for p in l1["edge"]])
    wd1 = jnp.stack([p["W1"][din1:] for p in l1["edge"]])
    b11 = jnp.stack([p["b1"] for p in l1["edge"]]).reshape(ET, 1, H)
    w21 = jnp.stack([p["W2"] for p in l1["edge"]])
    b21 = jnp.stack([p["b2"] for p in l1["edge"]]).reshape(ET, 1, H)
    w1n1 = jnp.stack([p["W1"] for p in l1["node"]])
    w2n1 = jnp.stack([p["W2"] for p in l1["node"]])
    dm1 = w1n1.shape[2]
    b1n1 = jnp.stack([p["b1"] for p in l1["node"]]).reshape(2, 1, dm1)
    b2n1 = jnp.stack([p["b2"] for p in l1["node"]]).reshape(2, 1, dm1)

    ps0, pd0 = _tc_pre(nf, ws0, wd0, b10)
    acc0, cnt = _sc_edge_pass(True)(
        ps0.reshape(ROWS, H), pd0.reshape(ROWS, H), src, dst, et)
    acc0r = acc0.reshape(NC, ROWS_PAD, H)[:, :ROWS].reshape(NC, ET, N_NODES, H)
    cntr = cnt.reshape(NC, ROWS_PAD, 16)[:, :ROWS].reshape(NC, ET, N_NODES, 16)
    ps1, pd1 = _tc_mid(acc0r, cntr, nt2, w20, b20, w1n0, b1n0, w2n0, b2n0,
                       ws1, wd1, b11)
    acc1 = _sc_edge_pass(False)(
        ps1.reshape(ROWS, H), pd1.reshape(ROWS, H), src, dst, et)
    acc1r = acc1.reshape(NC, ROWS_PAD, H)[:, :ROWS].reshape(NC, ET, N_NODES, H)
    return _tc_post(acc1r, cntr, nt2, w21, b21, w1n1, b1n1, w2n1, b2n1)


# speedup vs baseline: 16.6166x; 1.9010x over previous
import jax
import jax.numpy as jnp
from jax import lax
from jax.experimental import pallas as pl
from jax.experimental.pallas import tpu as pltpu
from jax.experimental.pallas import tpu_sc as plsc

N_NODES = 10000
N_EDGES = 320000
ET = 3
H = 32
NC, NS = 2, 16
NW = NC * NS
ROWS = ET * N_NODES
ROWS_PAD = 30720
RPT = ROWS_PAD // NS
ZROWS = 120
CHUNK = 80
EPW = N_EDGES // NW
NCHUNK = EPW // CHUNK
SB_EDGES = 2000
SB_CHUNKS = SB_EDGES // CHUNK
NSB = EPW // SB_EDGES
NB = 10
BN = N_NODES // NB


def _sc_edge_pass(with_count):
    def body(ps, pd, srcr, dstr, etr, *rest):
        if with_count:
            (acc_out, cnt_out, srcb, dstb, etb, idxs_t, idxd_t,
             av0, bv0, av1, bv1, stage,
             sa0, sb0, sa1, sb1, acc_s, onesv, cstage, cnt_s) = rest
        else:
            (acc_out, srcb, dstb, etb, idxs_t, idxd_t,
             av0, bv0, av1, bv1, stage,
             sa0, sb0, sa1, sb1, acc_s) = rest

        cid = lax.axis_index("c")
        sid = lax.axis_index("s")
        wid = sid * NC + cid
        tbase = sid * RPT
        ebase = wid * EPW
        zvec = jnp.zeros((16,), jnp.float32)

        def zrow(i, _):
            stage[i, pl.ds(0, 16)] = zvec
            stage[i, pl.ds(16, 16)] = zvec
            if with_count:
                cstage[i, pl.ds(0, 16)] = zvec
            return 0
        lax.fori_loop(0, ZROWS, zrow, 0)

        def zcp(j, _):
            b = tbase + j * ZROWS
            pltpu.sync_copy(stage, acc_s.at[pl.ds(b, ZROWS)])
            if with_count:
                pltpu.sync_copy(cstage, cnt_s.at[pl.ds(b, ZROWS)])
            return 0
        lax.fori_loop(0, RPT // ZROWS, zcp, 0)

        if with_count:
            iot = lax.iota(jnp.int32, 16)
            onevec = (1 - jnp.minimum(iot, 1)).astype(jnp.float32)

            def orow(i, _):
                onesv[i, pl.ds(0, 16)] = onevec
                return 0
            lax.fori_loop(0, CHUNK, orow, 0)

        plsc.subcore_barrier()

        def gather(j, a, b, sa, sb):
            c1 = pltpu.async_copy(ps.at[idxs_t.at[j]], a, sa)
            c2 = pltpu.async_copy(pd.at[idxd_t.at[j]], b, sb)
            return c1, c2

        def finish(j, a, b, sa, sb):
            pltpu.make_async_copy(ps.at[idxs_t.at[j]], a, sa).wait()
            pltpu.make_async_copy(pd.at[idxd_t.at[j]], b, sb).wait()
            for i in range(CHUNK):
                for h0 in (0, 16):
                    sl = pl.ds(h0, 16)
                    a[i, sl] = jnp.maximum(a[i, sl] + b[i, sl], zvec)
            pltpu.sync_copy(a, acc_s.at[idxd_t.at[j]], add=True)
            if with_count:
                pltpu.sync_copy(onesv, cnt_s.at[idxd_t.at[j]], add=True)

        def superblock(k, _):
            boff = ebase + k * SB_EDGES
            pltpu.sync_copy(srcr.at[pl.ds(boff, SB_EDGES)], srcb)
            pltpu.sync_copy(dstr.at[pl.ds(boff, SB_EDGES)], dstb)
            pltpu.sync_copy(etr.at[pl.ds(boff, SB_EDGES)], etb)

            def ibuild(j, _):
                for i in range(CHUNK // 16):
                    sl = pl.ds(j * CHUNK + i * 16, 16)
                    ebias = etb[sl] * N_NODES
                    idxs_t[j, pl.ds(i * 16, 16)] = ebias + srcb[sl]
                    idxd_t[j, pl.ds(i * 16, 16)] = ebias + dstb[sl]
                return 0
            lax.fori_loop(0, SB_CHUNKS, ibuild, 0)

            gather(0, av0, bv0, sa0, sb0)

            def pair(t, _):
                j = 2 * t
                gather(j + 1, av1, bv1, sa1, sb1)
                finish(j, av0, bv0, sa0, sb0)
                gather(j + 2, av0, bv0, sa0, sb0)
                finish(j + 1, av1, bv1, sa1, sb1)
                return 0
            lax.fori_loop(0, (SB_CHUNKS - 1) // 2, pair, 0)

            finish(SB_CHUNKS - 1, av0, bv0, sa0, sb0)
            return 0
        lax.fori_loop(0, NSB, superblock, 0)

        plsc.subcore_barrier()

        def wout(j, _):
            b = tbase + j * ZROWS
            pltpu.sync_copy(acc_s.at[pl.ds(b, ZROWS)], stage)
            pltpu.sync_copy(stage, acc_out.at[pl.ds(cid * ROWS_PAD + b, ZROWS)])
            if with_count:
                pltpu.sync_copy(cnt_s.at[pl.ds(b, ZROWS)], cstage)
                pltpu.sync_copy(cstage, cnt_out.at[pl.ds(cid * ROWS_PAD + b, ZROWS)])
            return 0
        lax.fori_loop(0, RPT // ZROWS, wout, 0)

    outs = [jax.ShapeDtypeStruct((NC * ROWS_PAD, H), jnp.float32)]
    scratch = [
        pltpu.VMEM((SB_EDGES,), jnp.int32),
        pltpu.VMEM((SB_EDGES,), jnp.int32),
        pltpu.VMEM((SB_EDGES,), jnp.int32),
        pltpu.VMEM((SB_CHUNKS, CHUNK), jnp.int32),
        pltpu.VMEM((SB_CHUNKS, CHUNK), jnp.int32),
        pltpu.VMEM((CHUNK, H), jnp.float32),
        pltpu.VMEM((CHUNK, H), jnp.float32),
        pltpu.VMEM((CHUNK, H), jnp.float32),
        pltpu.VMEM((CHUNK, H), jnp.float32),
        pltpu.VMEM((ZROWS, H), jnp.float32),
        pltpu.SemaphoreType.DMA,
        pltpu.SemaphoreType.DMA,
        pltpu.SemaphoreType.DMA,
        pltpu.SemaphoreType.DMA,
        pltpu.VMEM_SHARED((ROWS_PAD, H), jnp.float32),
    ]
    if with_count:
        outs.append(jax.ShapeDtypeStruct((NC * ROWS_PAD, 16), jnp.float32))
        scratch += [
            pltpu.VMEM((CHUNK, 16), jnp.float32),
            pltpu.VMEM((ZROWS, 16), jnp.float32),
            pltpu.VMEM_SHARED((ROWS_PAD, 16), jnp.float32),
        ]

    mesh = plsc.VectorSubcoreMesh(core_axis_name="c", subcore_axis_name="s")
    return pl.kernel(
        body,
        out_type=tuple(outs) if with_count else outs[0],
        scratch_types=scratch,
        mesh=mesh,
        compiler_params=pltpu.CompilerParams(use_tc_tiling_on_sc=False),
    )


def _tc_pre(nf, ws, wd, b1):
    din = nf.shape[1]

    def body(nf_ref, ws_ref, wd_ref, b1_ref, ps_ref, pd_ref):
        x = nf_ref[...]
        for e in range(ET):
            ps_ref[e] = jnp.dot(x, ws_ref[e], preferred_element_type=jnp.float32)
            pd_ref[e] = (jnp.dot(x, wd_ref[e], preferred_element_type=jnp.float32)
                         + b1_ref[e])

    return pl.pallas_call(
        body,
        grid=(NB,),
        in_specs=[
            pl.BlockSpec((BN, din), lambda g: (g, 0)),
            pl.BlockSpec((ET, din, H), lambda g: (0, 0, 0)),
            pl.BlockSpec((ET, din, H), lambda g: (0, 0, 0)),
            pl.BlockSpec((ET, 1, H), lambda g: (0, 0, 0)),
        ],
        out_specs=[
            pl.BlockSpec((ET, BN, H), lambda g: (0, g, 0)),
            pl.BlockSpec((ET, BN, H), lambda g: (0, g, 0)),
        ],
        out_shape=[
            jax.ShapeDtypeStruct((ET, N_NODES, H), jnp.float32),
            jax.ShapeDtypeStruct((ET, N_NODES, H), jnp.float32),
        ],
    )(nf, ws, wd, b1)


def _node_update(acc_ref, cnt_ref, nt_ref, w2_ref, b2_ref,
                 w1n_ref, b1n_ref, w2n_ref, b2n_ref):
    msgs = []
    for e in range(ET):
        s = acc_ref[0, e] + acc_ref[1, e]
        c = jnp.sum(cnt_ref[0, e] + cnt_ref[1, e], axis=-1,
                    keepdims=True)
        m = s / jnp.maximum(c, 1.0)
        ind = (c > 0.0).astype(jnp.float32)
        msgs.append(jnp.dot(m, w2_ref[e], preferred_element_type=jnp.float32)
                    + b2_ref[e] * ind)
    msg = jnp.concatenate(msgs, axis=-1)
    outs = []
    for i in range(2):
        hh = jnp.maximum(
            jnp.dot(msg, w1n_ref[i], preferred_element_type=jnp.float32)
            + b1n_ref[i], 0.0)
        outs.append(jnp.dot(hh, w2n_ref[i], preferred_element_type=jnp.float32)
                    + b2n_ref[i])
    ntv = nt_ref[...]
    return jnp.where(ntv == 1, outs[1], outs[0])


def _head_specs(dout):
    return [
        pl.BlockSpec((NC, ET, BN, H), lambda g: (0, 0, g, 0)),
        pl.BlockSpec((NC, ET, BN, 16), lambda g: (0, 0, g, 0)),
        pl.BlockSpec((BN, 1), lambda g: (g, 0)),
        pl.BlockSpec((ET, H, H), lambda g: (0, 0, 0)),
        pl.BlockSpec((ET, 1, H), lambda g: (0, 0, 0)),
        pl.BlockSpec((2, ET * H, dout), lambda g: (0, 0, 0)),
        pl.BlockSpec((2, 1, dout), lambda g: (0, 0, 0)),
        pl.BlockSpec((2, dout, dout), lambda g: (0, 0, 0)),
        pl.BlockSpec((2, 1, dout), lambda g: (0, 0, 0)),
    ]


def _tc_mid(acc, cnt, nt2, w2, b2, w1n, b1n, w2n, b2n, ws1, wd1, b11):
    dout = w1n.shape[2]

    def body(acc_ref, cnt_ref, nt_ref, w2_ref, b2_ref, w1n_ref, b1n_ref,
             w2n_ref, b2n_ref, ws1_ref, wd1_ref, b11_ref, ps_ref, pd_ref):
        x = _node_update(acc_ref, cnt_ref, nt_ref, w2_ref, b2_ref,
                         w1n_ref, b1n_ref, w2n_ref, b2n_ref)
        for e in range(ET):
            ps_ref[e] = jnp.dot(x, ws1_ref[e], preferred_element_type=jnp.float32)
            pd_ref[e] = (jnp.dot(x, wd1_ref[e], preferred_element_type=jnp.float32)
                         + b11_ref[e])

    return pl.pallas_call(
        body,
        grid=(NB,),
        in_specs=_head_specs(dout) + [
            pl.BlockSpec((ET, dout, H), lambda g: (0, 0, 0)),
            pl.BlockSpec((ET, dout, H), lambda g: (0, 0, 0)),
            pl.BlockSpec((ET, 1, H), lambda g: (0, 0, 0)),
        ],
        out_specs=[
            pl.BlockSpec((ET, BN, H), lambda g: (0, g, 0)),
            pl.BlockSpec((ET, BN, H), lambda g: (0, g, 0)),
        ],
        out_shape=[
            jax.ShapeDtypeStruct((ET, N_NODES, H), jnp.float32),
            jax.ShapeDtypeStruct((ET, N_NODES, H), jnp.float32),
        ],
    )(acc, cnt, nt2, w2, b2, w1n, b1n, w2n, b2n, ws1, wd1, b11)


def _tc_post(acc, cnt, nt2, w2, b2, w1n, b1n, w2n, b2n):
    dout = w1n.shape[2]

    def body(acc_ref, cnt_ref, nt_ref, w2_ref, b2_ref, w1n_ref, b1n_ref,
             w2n_ref, b2n_ref, out_ref):
        out_ref[...] = _node_update(acc_ref, cnt_ref, nt_ref, w2_ref, b2_ref,
                                    w1n_ref, b1n_ref, w2n_ref, b2n_ref)

    return pl.pallas_call(
        body,
        grid=(NB,),
        in_specs=_head_specs(dout),
        out_specs=pl.BlockSpec((BN, dout), lambda g: (g, 0)),
        out_shape=jax.ShapeDtypeStruct((N_NODES, dout), jnp.float32),
    )(acc, cnt, nt2, w2, b2, w1n, b1n, w2n, b2n)


def kernel(nf, edge_index, edge_type, node_type, params):
    src = edge_index[0].astype(jnp.int32)
    dst = edge_index[1].astype(jnp.int32)
    et = edge_type.astype(jnp.int32)
    nt2 = node_type.astype(jnp.int32).reshape(N_NODES, 1)

    l0, l1 = params
    din0 = nf.shape[1]
    ws0 = jnp.stack([p["W1"][:din0] for p in l0["edge"]])
    wd0 = jnp.stack([p["W1"][din0:] for p in l0["edge"]])
    b10 = jnp.stack([p["b1"] for p in l0["edge"]]).reshape(ET, 1, H)
    w20 = jnp.stack([p["W2"] for p in l0["edge"]])
    b20 = jnp.stack([p["b2"] for p in l0["edge"]]).reshape(ET, 1, H)
    w1n0 = jnp.stack([p["W1"] for p in l0["node"]])
    w2n0 = jnp.stack([p["W2"] for p in l0["node"]])
    dm0 = w1n0.shape[2]
    b1n0 = jnp.stack([p["b1"] for p in l0["node"]]).reshape(2, 1, dm0)
    b2n0 = jnp.stack([p["b2"] for p in l0["node"]]).reshape(2, 1, dm0)
    din1 = dm0
    ws1 = jnp.stack([p["W1"][:din1] for p in l1["edge"]])
    wd1 = jnp.stack([p["W1"][din1:] for p in l1["edge"]])
    b11 = jnp.stack([p["b1"] for p in l1["edge"]]).reshape(ET, 1, H)
    w21 = jnp.stack([p["W2"] for p in l1["edge"]])
    b21 = jnp.stack([p["b2"] for p in l1["edge"]]).reshape(ET, 1, H)
    w1n1 = jnp.stack([p["W1"] for p in l1["node"]])
    w2n1 = jnp.stack([p["W2"] for p in l1["node"]])
    dm1 = w1n1.shape[2]
    b1n1 = jnp.stack([p["b1"] for p in l1["node"]]).reshape(2, 1, dm1)
    b2n1 = jnp.stack([p["b2"] for p in l1["node"]]).reshape(2, 1, dm1)

    ps0, pd0 = _tc_pre(nf, ws0, wd0, b10)
    acc0, cnt = _sc_edge_pass(True)(
        ps0.reshape(ROWS, H), pd0.reshape(ROWS, H), src, dst, et)
    acc0r = acc0.reshape(NC, ROWS_PAD, H)[:, :ROWS].reshape(NC, ET, N_NODES, H)
    cntr = cnt.reshape(NC, ROWS_PAD, 16)[:, :ROWS].reshape(NC, ET, N_NODES, 16)
    ps1, pd1 = _tc_mid(acc0r, cntr, nt2, w20, b20, w1n0, b1n0, w2n0, b2n0,
                       ws1, wd1, b11)
    acc1 = _sc_edge_pass(False)(
        ps1.reshape(ROWS, H), pd1.reshape(ROWS, H), src, dst, et)
    acc1r = acc1.reshape(NC, ROWS_PAD, H)[:, :ROWS].reshape(NC, ET, N_NODES, H)
    return _tc_post(acc1r, cntr, nt2, w21, b21, w1n1, b1n1, w2n1, b2n1)


# speedup vs baseline: 18.0702x; 1.0875x over previous
import jax
import jax.numpy as jnp
from jax import lax
from jax.experimental import pallas as pl
from jax.experimental.pallas import tpu as pltpu
from jax.experimental.pallas import tpu_sc as plsc

N_NODES = 10000
N_EDGES = 320000
ET = 3
H = 32
NC, NS = 2, 16
NW = NC * NS
ROWS = ET * N_NODES
ROWS_PAD = 30720
RPT = ROWS_PAD // NS
ZROWS = 120
CHUNK = 80
EPW = N_EDGES // NW
NCHUNK = EPW // CHUNK
SB_EDGES = 2000
SB_CHUNKS = SB_EDGES // CHUNK
NSB = EPW // SB_EDGES
NB = 10
BN = N_NODES // NB
QS = 2504
NPAD = 4 * QS
TROWS = ET * NPAD


def _sc_edge_pass(with_count):
    def body(ps, pd, srcr, dstr, etr, *rest):
        if with_count:
            (acc_out, cnt_out, srcb, dstb, etb, idxs_t, idxd_t, idxw_t,
             av0, bv0, av1, bv1, stage,
             sa0, sb0, sa1, sb1, acc_s, onesv, cstage, cnt_s) = rest
        else:
            (acc_out, srcb, dstb, etb, idxs_t, idxd_t, idxw_t,
             av0, bv0, av1, bv1, stage,
             sa0, sb0, sa1, sb1, acc_s) = rest

        cid = lax.axis_index("c")
        sid = lax.axis_index("s")
        wid = sid * NC + cid
        tbase = sid * RPT
        ebase = wid * EPW
        zvec = jnp.zeros((16,), jnp.float32)

        def zrow(i, _):
            stage[i, pl.ds(0, 16)] = zvec
            stage[i, pl.ds(16, 16)] = zvec
            if with_count:
                cstage[i, pl.ds(0, 16)] = zvec
            return 0
        lax.fori_loop(0, ZROWS, zrow, 0)

        def zcp(j, _):
            b = tbase + j * ZROWS
            pltpu.sync_copy(stage, acc_s.at[pl.ds(b, ZROWS)])
            if with_count:
                pltpu.sync_copy(cstage, cnt_s.at[pl.ds(b, ZROWS)])
            return 0
        lax.fori_loop(0, RPT // ZROWS, zcp, 0)

        if with_count:
            iot = lax.iota(jnp.int32, 16)
            onevec = (1 - jnp.minimum(iot, 1)).astype(jnp.float32)

            def orow(i, _):
                onesv[i, pl.ds(0, 16)] = onevec
                return 0
            lax.fori_loop(0, CHUNK, orow, 0)

        plsc.subcore_barrier()

        def gather(j, a, b, sa, sb):
            c1 = pltpu.async_copy(ps.at[idxs_t.at[j]], a, sa)
            c2 = pltpu.async_copy(pd.at[idxd_t.at[j]], b, sb)
            return c1, c2

        def finish(j, a, b, sa, sb):
            pltpu.make_async_copy(ps.at[idxs_t.at[j]], a, sa).wait()
            pltpu.make_async_copy(pd.at[idxd_t.at[j]], b, sb).wait()
            for i in range(CHUNK):
                for h0 in (0, 16):
                    sl = pl.ds(h0, 16)
                    a[i, sl] = jnp.maximum(a[i, sl] + b[i, sl], zvec)
            pltpu.sync_copy(a, acc_s.at[idxw_t.at[j]], add=True)
            if with_count:
                pltpu.sync_copy(onesv, cnt_s.at[idxw_t.at[j]], add=True)

        def superblock(k, _):
            boff = ebase + k * SB_EDGES
            pltpu.sync_copy(srcr.at[pl.ds(boff, SB_EDGES)], srcb)
            pltpu.sync_copy(dstr.at[pl.ds(boff, SB_EDGES)], dstb)
            pltpu.sync_copy(etr.at[pl.ds(boff, SB_EDGES)], etb)

            def ibuild(j, _):
                for i in range(CHUNK // 16):
                    sl = pl.ds(j * CHUNK + i * 16, 16)
                    et16 = etb[sl]
                    s16 = srcb[sl]
                    d16 = dstb[sl]
                    qs_ = (s16 * 13401) >> 25
                    qd_ = (d16 * 13401) >> 25
                    tb = et16 * NPAD
                    idxs_t[j, pl.ds(i * 16, 16)] = tb + (s16 - qs_ * QS) * 4 + qs_
                    idxd_t[j, pl.ds(i * 16, 16)] = tb + (d16 - qd_ * QS) * 4 + qd_
                    idxw_t[j, pl.ds(i * 16, 16)] = et16 * N_NODES + d16
                return 0
            lax.fori_loop(0, SB_CHUNKS, ibuild, 0)

            gather(0, av0, bv0, sa0, sb0)

            def pair(t, _):
                j = 2 * t
                gather(j + 1, av1, bv1, sa1, sb1)
                finish(j, av0, bv0, sa0, sb0)
                gather(j + 2, av0, bv0, sa0, sb0)
                finish(j + 1, av1, bv1, sa1, sb1)
                return 0
            lax.fori_loop(0, (SB_CHUNKS - 1) // 2, pair, 0)

            finish(SB_CHUNKS - 1, av0, bv0, sa0, sb0)
            return 0
        lax.fori_loop(0, NSB, superblock, 0)

        plsc.subcore_barrier()

        def wout(j, _):
            b = tbase + j * ZROWS
            pltpu.sync_copy(acc_s.at[pl.ds(b, ZROWS)], stage)
            pltpu.sync_copy(stage, acc_out.at[pl.ds(cid * ROWS_PAD + b, ZROWS)])
            if with_count:
                pltpu.sync_copy(cnt_s.at[pl.ds(b, ZROWS)], cstage)
                pltpu.sync_copy(cstage, cnt_out.at[pl.ds(cid * ROWS_PAD + b, ZROWS)])
            return 0
        lax.fori_loop(0, RPT // ZROWS, wout, 0)

    outs = [jax.ShapeDtypeStruct((NC * ROWS_PAD, H), jnp.float32)]
    scratch = [
        pltpu.VMEM((SB_EDGES,), jnp.int32),
        pltpu.VMEM((SB_EDGES,), jnp.int32),
        pltpu.VMEM((SB_EDGES,), jnp.int32),
        pltpu.VMEM((SB_CHUNKS, CHUNK), jnp.int32),
        pltpu.VMEM((SB_CHUNKS, CHUNK), jnp.int32),
        pltpu.VMEM((SB_CHUNKS, CHUNK), jnp.int32),
        pltpu.VMEM((CHUNK, H), jnp.float32),
        pltpu.VMEM((CHUNK, H), jnp.float32),
        pltpu.VMEM((CHUNK, H), jnp.float32),
        pltpu.VMEM((CHUNK, H), jnp.float32),
        pltpu.VMEM((ZROWS, H), jnp.float32),
        pltpu.SemaphoreType.DMA,
        pltpu.SemaphoreType.DMA,
        pltpu.SemaphoreType.DMA,
        pltpu.SemaphoreType.DMA,
        pltpu.VMEM_SHARED((ROWS_PAD, H), jnp.float32),
    ]
    if with_count:
        outs.append(jax.ShapeDtypeStruct((NC * ROWS_PAD, 16), jnp.float32))
        scratch += [
            pltpu.VMEM((CHUNK, 16), jnp.float32),
            pltpu.VMEM((ZROWS, 16), jnp.float32),
            pltpu.VMEM_SHARED((ROWS_PAD, 16), jnp.float32),
        ]

    mesh = plsc.VectorSubcoreMesh(core_axis_name="c", subcore_axis_name="s")
    return pl.kernel(
        body,
        out_type=tuple(outs) if with_count else outs[0],
        scratch_types=scratch,
        mesh=mesh,
        compiler_params=pltpu.CompilerParams(use_tc_tiling_on_sc=False),
    )


def _tc_proj(nf, ws, wd, b1):
    din = nf.shape[1]
    n = nf.shape[0]

    def body(nf_ref, ws_ref, wd_ref, b1_ref, ps_ref, pd_ref):
        xs = []
        for q in range(4):
            lo = q * QS
            if lo + QS <= n:
                xs.append(nf_ref[pl.ds(lo, QS), :])
            else:
                tail = nf_ref[pl.ds(lo, n - lo), :]
                xs.append(jnp.concatenate(
                    [tail, jnp.zeros((lo + QS - n, din), jnp.float32)], axis=0))
        for e in range(ET):
            a = [jnp.dot(x, ws_ref[e], preferred_element_type=jnp.float32)
                 for x in xs]
            d = [jnp.dot(x, wd_ref[e], preferred_element_type=jnp.float32)
                 + b1_ref[e] for x in xs]
            ps_ref[e] = jnp.concatenate(a, axis=-1)
            pd_ref[e] = jnp.concatenate(d, axis=-1)

    return pl.pallas_call(
        body,
        out_shape=[
            jax.ShapeDtypeStruct((ET, QS, 128), jnp.float32),
            jax.ShapeDtypeStruct((ET, QS, 128), jnp.float32),
        ],
    )(nf, ws, wd, b1)


def _node_update(acc_ref, cnt_ref, nt_ref, w2_ref, b2_ref,
                 w1n_ref, b1n_ref, w2n_ref, b2n_ref):
    msgs = []
    for e in range(ET):
        s = acc_ref[0, e] + acc_ref[1, e]
        c = jnp.sum(cnt_ref[0, e] + cnt_ref[1, e], axis=-1,
                    keepdims=True)
        m = s / jnp.maximum(c, 1.0)
        ind = (c > 0.0).astype(jnp.float32)
        msgs.append(jnp.dot(m, w2_ref[e], preferred_element_type=jnp.float32)
                    + b2_ref[e] * ind)
    msg = jnp.concatenate(msgs, axis=-1)
    outs = []
    for i in range(2):
        hh = jnp.maximum(
            jnp.dot(msg, w1n_ref[i], preferred_element_type=jnp.float32)
            + b1n_ref[i], 0.0)
        outs.append(jnp.dot(hh, w2n_ref[i], preferred_element_type=jnp.float32)
                    + b2n_ref[i])
    ntv = nt_ref[...]
    return jnp.where(ntv == 1, outs[1], outs[0])


def _head_specs(dout):
    return [
        pl.BlockSpec((NC, ET, BN, H), lambda g: (0, 0, g, 0)),
        pl.BlockSpec((NC, ET, BN, 16), lambda g: (0, 0, g, 0)),
        pl.BlockSpec((BN, 1), lambda g: (g, 0)),
        pl.BlockSpec((ET, H, H), lambda g: (0, 0, 0)),
        pl.BlockSpec((ET, 1, H), lambda g: (0, 0, 0)),
        pl.BlockSpec((2, ET * H, dout), lambda g: (0, 0, 0)),
        pl.BlockSpec((2, 1, dout), lambda g: (0, 0, 0)),
        pl.BlockSpec((2, dout, dout), lambda g: (0, 0, 0)),
        pl.BlockSpec((2, 1, dout), lambda g: (0, 0, 0)),
    ]


def _tc_mid(acc, cnt, nt2, w2, b2, w1n, b1n, w2n, b2n):
    dout = w1n.shape[2]

    def body(acc_ref, cnt_ref, nt_ref, w2_ref, b2_ref, w1n_ref, b1n_ref,
             w2n_ref, b2n_ref, out_ref):
        out_ref[...] = _node_update(acc_ref, cnt_ref, nt_ref, w2_ref, b2_ref,
                                    w1n_ref, b1n_ref, w2n_ref, b2n_ref)

    return pl.pallas_call(
        body,
        grid=(NB,),
        in_specs=_head_specs(dout),
        out_specs=pl.BlockSpec((BN, dout), lambda g: (g, 0)),
        out_shape=jax.ShapeDtypeStruct((N_NODES, dout), jnp.float32),
    )(acc, cnt, nt2, w2, b2, w1n, b1n, w2n, b2n)


def _tc_post(acc, cnt, nt2, w2, b2, w1n, b1n, w2n, b2n):
    dout = w1n.shape[2]

    def body(acc_ref, cnt_ref, nt_ref, w2_ref, b2_ref, w1n_ref, b1n_ref,
             w2n_ref, b2n_ref, out_ref):
        out_ref[...] = _node_update(acc_ref, cnt_ref, nt_ref, w2_ref, b2_ref,
                                    w1n_ref, b1n_ref, w2n_ref, b2n_ref)

    return pl.pallas_call(
        body,
        grid=(NB,),
        in_specs=_head_specs(dout),
        out_specs=pl.BlockSpec((BN, dout), lambda g: (g, 0)),
        out_shape=jax.ShapeDtypeStruct((N_NODES, dout), jnp.float32),
    )(acc, cnt, nt2, w2, b2, w1n, b1n, w2n, b2n)


def kernel(nf, edge_index, edge_type, node_type, params):
    src = edge_index[0].astype(jnp.int32)
    dst = edge_index[1].astype(jnp.int32)
    et = edge_type.astype(jnp.int32)
    nt2 = node_type.astype(jnp.int32).reshape(N_NODES, 1)

    l0, l1 = params
    din0 = nf.shape[1]
    ws0 = jnp.stack([p["W1"][:din0] for p in l0["edge"]])
    wd0 = jnp.stack([p["W1"][din0:] for p in l0["edge"]])
    b10 = jnp.stack([p["b1"] for p in l0["edge"]]).reshape(ET, 1, H)
    w20 = jnp.stack([p["W2"] for p in l0["edge"]])
    b20 = jnp.stack([p["b2"] for p in l0["edge"]]).reshape(ET, 1, H)
    w1n0 = jnp.stack([p["W1"] for p in l0["node"]])
    w2n0 = jnp.stack([p["W2"] for p in l0["node"]])
    dm0 = w1n0.shape[2]
    b1n0 = jnp.stack([p["b1"] for p in l0["node"]]).reshape(2, 1, dm0)
    b2n0 = jnp.stack([p["b2"] for p in l0["node"]]).reshape(2, 1, dm0)
    din1 = dm0
    ws1 = jnp.stack([p["W1"][:din1] for p in l1["edge"]])
    wd1 = jnp.stack([p["W1"][din1:] for p in l1["edge"]])
    b11 = jnp.stack([p["b1"] for p in l1["edge"]]).reshape(ET, 1, H)
    w21 = jnp.stack([p["W2"] for p in l1["edge"]])
    b21 = jnp.stack([p["b2"] for p in l1["edge"]]).reshape(ET, 1, H)
    w1n1 = jnp.stack([p["W1"] for p in l1["node"]])
    w2n1 = jnp.stack([p["W2"] for p in l1["node"]])
    dm1 = w1n1.shape[2]
    b1n1 = jnp.stack([p["b1"] for p in l1["node"]]).reshape(2, 1, dm1)
    b2n1 = jnp.stack([p["b2"] for p in l1["node"]]).reshape(2, 1, dm1)

    ps0, pd0 = _tc_proj(nf, ws0, wd0, b10)
    acc0, cnt = _sc_edge_pass(True)(
        ps0.reshape(TROWS, H), pd0.reshape(TROWS, H), src, dst, et)
    acc0r = acc0.reshape(NC, ROWS_PAD, H)[:, :ROWS].reshape(NC, ET, N_NODES, H)
    cntr = cnt.reshape(NC, ROWS_PAD, 16)[:, :ROWS].reshape(NC, ET, N_NODES, 16)
    nf1 = _tc_mid(acc0r, cntr, nt2, w20, b20, w1n0, b1n0, w2n0, b2n0)
    ps1, pd1 = _tc_proj(nf1, ws1, wd1, b11)
    acc1 = _sc_edge_pass(False)(
        ps1.reshape(TROWS, H), pd1.reshape(TROWS, H), src, dst, et)
    acc1r = acc1.reshape(NC, ROWS_PAD, H)[:, :ROWS].reshape(NC, ET, N_NODES, H)
    return _tc_post(acc1r, cntr, nt2, w21, b21, w1n1, b1n1, w2n1, b2n1)
